# Initial kernel scaffold; baseline (speedup 1.0000x reference)
import jax
import jax.numpy as jnp
from jax.experimental import pallas as pl


def kernel(loc_data, conf_data, locT, confT, priors, targets):
    raise NotImplementedError("write your pallas kernel here")


# speedup vs baseline: 2.6891x; 2.6891x over previous
import functools

import jax
import jax.numpy as jnp
from jax.experimental import pallas as pl
from jax.experimental.pallas import tpu as pltpu

_THRESH = 0.5
_MAX_FINITE_BITS = 0x7F7FFFFF


def _smooth_l1(x):
    ax = jnp.abs(x)
    return jnp.where(ax < 1.0, 0.5 * x * x, ax - 0.5)


def _mbody(locR_ref, conf_ref, locTR_ref, confT_ref, priT_ref, tgt_ref,
           out_ref, *, hint_denom):
    b = pl.program_id(0)
    nb = pl.num_programs(0)
    P = conf_ref.shape[1]
    C = conf_ref.shape[2]
    O = tgt_ref.shape[1]

    cf = conf_ref[0]
    cT = confT_ref[0]
    ld = locR_ref[0]
    lt = locTR_ref[0]
    pri = priT_ref[...]
    tg = tgt_ref[0]

    pcx = pri[0:1, :]
    pcy = pri[1:2, :]
    pw = pri[2:3, :]
    ph = pri[3:4, :]
    bx1 = pcx - pw * 0.5
    by1 = pcy - ph * 0.5
    bx2 = pcx + pw * 0.5
    by2 = pcy + ph * 0.5
    area_b = (bx2 - bx1) * (by2 - by1)

    tx1 = tg[:, 0:1]
    ty1 = tg[:, 1:2]
    tx2 = tg[:, 2:3]
    ty2 = tg[:, 3:4]
    lab = tg[:, 4:5]

    iw = jnp.maximum(jnp.minimum(tx2, bx2) - jnp.maximum(tx1, bx1), 0.0)
    ih = jnp.maximum(jnp.minimum(ty2, by2) - jnp.maximum(ty1, by1), 0.0)
    inter = iw * ih
    area_a = (tx2 - tx1) * (ty2 - ty1)
    ov = inter / (area_a + area_b - inter)

    bto = jnp.max(ov, axis=0, keepdims=True)
    rows = jax.lax.broadcasted_iota(jnp.int32, (O, P), 0)
    bti = jnp.min(jnp.where(ov == bto, rows, O), axis=0, keepdims=True)

    bpo = jnp.max(ov, axis=1, keepdims=True)
    cols = jax.lax.broadcasted_iota(jnp.int32, (O, P), 1)
    bpi = jnp.min(jnp.where(ov == bpo, cols, P), axis=1, keepdims=True)

    lane_p = jax.lax.broadcasted_iota(jnp.int32, (1, P), 1)
    for j in range(O):
        mj = lane_p == bpi[j:j + 1, 0:1]
        bto = jnp.where(mj, 2.0, bto)
        bti = jnp.where(mj, j, bti)

    labg = jnp.zeros((1, P), jnp.float32)
    mx1 = jnp.zeros((1, P), jnp.float32)
    my1 = jnp.zeros((1, P), jnp.float32)
    mx2 = jnp.zeros((1, P), jnp.float32)
    my2 = jnp.zeros((1, P), jnp.float32)
    for j in range(O):
        e = bti == j
        labg = jnp.where(e, lab[j:j + 1, 0:1], labg)
        mx1 = jnp.where(e, tx1[j:j + 1, 0:1], mx1)
        my1 = jnp.where(e, ty1[j:j + 1, 0:1], my1)
        mx2 = jnp.where(e, tx2[j:j + 1, 0:1], mx2)
        my2 = jnp.where(e, ty2[j:j + 1, 0:1], my2)

    conf_row = jnp.where(bto < _THRESH, 0, labg.astype(jnp.int32) + 1)
    posr = conf_row > 0
    posf = posr.astype(jnp.float32)
    n_pos = jnp.sum(posf)

    g_cx = ((mx1 + mx2) * 0.5 - pcx) / (0.1 * pw)
    g_cy = ((my1 + my2) * 0.5 - pcy) / (0.1 * ph)
    g_w = jnp.log((mx2 - mx1) / pw) / 0.2
    g_h = jnp.log((my2 - my1) / ph) / 0.2

    l_l = jnp.float32(0.0)
    ss_s = jnp.float32(0.0)
    ss_t = jnp.float32(0.0)
    for k, g in enumerate((g_cx, g_cy, g_w, g_h)):
        ds = ld[k:k + 1, :] - g
        dt = lt[k:k + 1, :] - g
        l_l = l_l + jnp.sum(_smooth_l1(ds) * posf)
        ss_s = ss_s + jnp.sum(ds * ds * posf)
        ss_t = ss_t + jnp.sum(dt * dt * posf)

    hint = jnp.sum((cf - cT) ** 2)
    m = jnp.max(cf, axis=1, keepdims=True)
    se = jnp.sum(jnp.exp(cf - m), axis=1, keepdims=True)
    ct_col = conf_row.reshape(P, 1)
    ccols = jax.lax.broadcasted_iota(jnp.int32, (P, C), 1)
    gathered = jnp.sum(jnp.where(ccols == ct_col, cf, 0.0), axis=1,
                       keepdims=True)
    se_row = se.reshape(1, P)
    mg_row = (m - gathered).reshape(1, P)
    ce_row = jnp.log(se_row) + mg_row
    lossc = jnp.where(posr, 0.0, ce_row)

    bits = jax.lax.bitcast_convert_type(lossc, jnp.int32)
    K = jnp.minimum(3 * jnp.sum(posr.astype(jnp.int32)), P - 1)
    lo = jnp.int32(0)
    hi = jnp.int32(_MAX_FINITE_BITS)
    for _ in range(31):
        mid = lo + ((hi - lo + 1) >> 1)
        c = jnp.sum(jnp.where(bits >= mid, 1, 0))
        ge = c >= K
        lo = jnp.where(ge, mid, lo)
        hi = jnp.where(ge, hi, mid - 1)
    vb = lo
    cnt_gt = jnp.sum(jnp.where(bits > vb, 1, 0))
    need = K - cnt_gt
    tie = bits == vb
    lo2 = jnp.int32(0)
    hi2 = jnp.int32(P)
    for _ in range(14):
        mid2 = (lo2 + hi2) >> 1
        c2 = jnp.sum(jnp.where(tie & (lane_p < mid2), 1, 0))
        ok = c2 >= need
        hi2 = jnp.where(ok, mid2, hi2)
        lo2 = jnp.where(ok, lo2, mid2 + 1)
    neg = (bits > vb) | (tie & (lane_p < lo2))
    sel = (posr | neg).astype(jnp.float32)
    ce_sel = jnp.sum(ce_row * sel)

    lane128 = jax.lax.broadcasted_iota(jnp.int32, (1, 128), 1)

    def put(k, v):
        return jnp.where(lane128 == k, v, 0.0)

    vals = (put(0, n_pos) + put(1, l_l) + put(2, ss_s) + put(3, ss_t)
            + put(4, ce_sel) + put(5, hint))
    acc = jnp.where(b == 0, vals, out_ref[...] + vals)

    def get(k):
        return jnp.sum(jnp.where(lane128 == k, acc, 0.0))

    Nf = get(0)
    llT = get(1)
    mse_s = get(2) / (Nf * 4.0)
    mse_t = get(3) / (Nf * 4.0)
    ceT = get(4)
    hintT = get(5)
    lbr = jnp.where(mse_s > mse_t, 0.5 * mse_s, 0.0)
    o1 = (ceT + llT + lbr) / Nf + 0.5 * hintT / hint_denom
    o2 = (ceT + llT) / Nf
    acc = jnp.where(b == nb - 1, acc + put(6, o1) + put(7, o2), acc)
    out_ref[...] = acc


def kernel(loc_data, conf_data, locT, confT, priors, targets):
    B, P, C = conf_data.shape
    O = targets.shape[1]
    locR = jnp.transpose(loc_data, (0, 2, 1))
    locTR = jnp.transpose(locT, (0, 2, 1))
    priT = jnp.transpose(priors, (1, 0))
    body = functools.partial(_mbody, hint_denom=float(B * P * C))
    res = pl.pallas_call(
        body,
        grid=(B,),
        in_specs=[
            pl.BlockSpec((1, 4, P), lambda b: (b, 0, 0)),
            pl.BlockSpec((1, P, C), lambda b: (b, 0, 0)),
            pl.BlockSpec((1, 4, P), lambda b: (b, 0, 0)),
            pl.BlockSpec((1, P, C), lambda b: (b, 0, 0)),
            pl.BlockSpec((4, P), lambda b: (0, 0)),
            pl.BlockSpec((1, O, 5), lambda b: (b, 0, 0)),
        ],
        out_specs=pl.BlockSpec((1, 128), lambda b: (0, 0)),
        out_shape=jax.ShapeDtypeStruct((1, 128), jnp.float32),
        compiler_params=pltpu.CompilerParams(
            dimension_semantics=("arbitrary",)),
    )(locR, conf_data, locTR, confT, priT, targets)
    return (res[0, 6], res[0, 7])


# speedup vs baseline: 2.7534x; 1.0239x over previous
import functools

import jax
import jax.numpy as jnp
from jax.experimental import pallas as pl
from jax.experimental.pallas import tpu as pltpu

_THRESH = 0.5
_MAX_FINITE_BITS = 0x7F7FFFFF


def _smooth_l1(x):
    ax = jnp.abs(x)
    return jnp.where(ax < 1.0, 0.5 * x * x, ax - 0.5)


def _one_image(cf, cT, ld, lt, pri, tg):
    P = cf.shape[0]
    C = cf.shape[1]
    O = tg.shape[0]

    pcx = pri[0:1, :]
    pcy = pri[1:2, :]
    pw = pri[2:3, :]
    ph = pri[3:4, :]
    bx1 = pcx - pw * 0.5
    by1 = pcy - ph * 0.5
    bx2 = pcx + pw * 0.5
    by2 = pcy + ph * 0.5
    area_b = (bx2 - bx1) * (by2 - by1)

    tx1 = tg[:, 0:1]
    ty1 = tg[:, 1:2]
    tx2 = tg[:, 2:3]
    ty2 = tg[:, 3:4]
    lab = tg[:, 4:5]

    iw = jnp.maximum(jnp.minimum(tx2, bx2) - jnp.maximum(tx1, bx1), 0.0)
    ih = jnp.maximum(jnp.minimum(ty2, by2) - jnp.maximum(ty1, by1), 0.0)
    inter = iw * ih
    area_a = (tx2 - tx1) * (ty2 - ty1)
    ov = inter / (area_a + area_b - inter)

    bto = jnp.max(ov, axis=0, keepdims=True)
    rows = jax.lax.broadcasted_iota(jnp.int32, (O, P), 0)
    bti = jnp.min(jnp.where(ov == bto, rows, O), axis=0, keepdims=True)

    bpo = jnp.max(ov, axis=1, keepdims=True)
    cols = jax.lax.broadcasted_iota(jnp.int32, (O, P), 1)
    bpi = jnp.min(jnp.where(ov == bpo, cols, P), axis=1, keepdims=True)

    lane_p = jax.lax.broadcasted_iota(jnp.int32, (1, P), 1)
    for j in range(O):
        mj = lane_p == bpi[j:j + 1, 0:1]
        bto = jnp.where(mj, 2.0, bto)
        bti = jnp.where(mj, j, bti)

    labg = jnp.zeros((1, P), jnp.float32)
    mx1 = jnp.zeros((1, P), jnp.float32)
    my1 = jnp.zeros((1, P), jnp.float32)
    mx2 = jnp.zeros((1, P), jnp.float32)
    my2 = jnp.zeros((1, P), jnp.float32)
    for j in range(O):
        e = bti == j
        labg = jnp.where(e, lab[j:j + 1, 0:1], labg)
        mx1 = jnp.where(e, tx1[j:j + 1, 0:1], mx1)
        my1 = jnp.where(e, ty1[j:j + 1, 0:1], my1)
        mx2 = jnp.where(e, tx2[j:j + 1, 0:1], mx2)
        my2 = jnp.where(e, ty2[j:j + 1, 0:1], my2)

    conf_row = jnp.where(bto < _THRESH, 0, labg.astype(jnp.int32) + 1)
    posr = conf_row > 0
    posf = posr.astype(jnp.float32)
    n_pos = jnp.sum(posf)

    g_cx = ((mx1 + mx2) * 0.5 - pcx) / (0.1 * pw)
    g_cy = ((my1 + my2) * 0.5 - pcy) / (0.1 * ph)
    g_w = jnp.log((mx2 - mx1) / pw) / 0.2
    g_h = jnp.log((my2 - my1) / ph) / 0.2

    l_l = jnp.float32(0.0)
    ss_s = jnp.float32(0.0)
    ss_t = jnp.float32(0.0)
    for k, g in enumerate((g_cx, g_cy, g_w, g_h)):
        ds = ld[k:k + 1, :] - g
        dt = lt[k:k + 1, :] - g
        l_l = l_l + jnp.sum(_smooth_l1(ds) * posf)
        ss_s = ss_s + jnp.sum(ds * ds * posf)
        ss_t = ss_t + jnp.sum(dt * dt * posf)

    hint = jnp.sum((cf - cT) ** 2)
    m = jnp.max(cf, axis=1, keepdims=True)
    se = jnp.sum(jnp.exp(cf - m), axis=1, keepdims=True)
    ct_col = conf_row.reshape(P, 1)
    ccols = jax.lax.broadcasted_iota(jnp.int32, (P, C), 1)
    gathered = jnp.sum(jnp.where(ccols == ct_col, cf, 0.0), axis=1,
                       keepdims=True)
    se_row = se.reshape(1, P)
    mg_row = (m - gathered).reshape(1, P)
    ce_row = jnp.log(se_row) + mg_row
    lossc = jnp.where(posr, 0.0, ce_row)

    bits = jax.lax.bitcast_convert_type(lossc, jnp.int32)
    sbits = bits ^ jnp.int32(-2 ** 31)
    K = jnp.minimum(3 * jnp.sum(posr.astype(jnp.int32)), P - 1)
    lo = jnp.int32(-2 ** 31)
    w = _MAX_FINITE_BITS + 1
    for _ in range(11):
        step = -(-w // 8)
        reds = []
        for j in range(1, 8, 2):
            ga = jnp.where(sbits >= lo + jnp.int32(j * step), 1, 0)
            if j + 1 < 8:
                ga = ga + jnp.where(
                    sbits >= lo + jnp.int32((j + 1) * step), 1 << 16, 0)
            reds.append(jnp.sum(ga))
        jmax = jnp.int32(0)
        for i, r in enumerate(reds):
            jmax = jmax + ((r & 0xFFFF) >= K).astype(jnp.int32)
            if 2 * i + 2 < 8:
                jmax = jmax + ((r >> 16) >= K).astype(jnp.int32)
        lo = lo + jmax * jnp.int32(step)
        w = step
    vb = lo
    cnt_gt = jnp.sum(jnp.where(sbits > vb, 1, 0))
    need = K - cnt_gt
    tie = sbits == vb
    lo2 = jnp.int32(0)
    w2 = 16384
    for _ in range(7):
        st = w2 // 4
        g1 = jnp.where(tie & (lane_p < lo2 + jnp.int32(st)), 1, 0)
        g1 = g1 + jnp.where(tie & (lane_p < lo2 + jnp.int32(2 * st)),
                            1 << 16, 0)
        r12 = jnp.sum(g1)
        c3 = jnp.sum(jnp.where(tie & (lane_p < lo2 + jnp.int32(3 * st)), 1, 0))
        jbel = ((r12 & 0xFFFF) < need).astype(jnp.int32)
        jbel = jbel + ((r12 >> 16) < need).astype(jnp.int32)
        jbel = jbel + (c3 < need).astype(jnp.int32)
        lo2 = lo2 + jbel * jnp.int32(st)
        w2 = st
    cfin = jnp.sum(jnp.where(tie & (lane_p < lo2), 1, 0))
    lo2 = jnp.where(cfin >= need, lo2, lo2 + 1)
    neg = (sbits > vb) | (tie & (lane_p < lo2))
    sel = (posr | neg).astype(jnp.float32)
    ce_sel = jnp.sum(ce_row * sel)
    return n_pos, l_l, ss_s, ss_t, ce_sel, hint


def _mbody(locR_ref, conf_ref, locTR_ref, confT_ref, priT_ref, tgt_ref,
           out_ref, *, hint_denom, imgs):
    b = pl.program_id(0)
    nb = pl.num_programs(0)
    pri = priT_ref[...]

    n_pos = l_l = ss_s = ss_t = ce_sel = hint = jnp.float32(0.0)
    for i in range(imgs):
        r = _one_image(conf_ref[i], confT_ref[i], locR_ref[i], locTR_ref[i],
                       pri, tgt_ref[i])
        n_pos = n_pos + r[0]
        l_l = l_l + r[1]
        ss_s = ss_s + r[2]
        ss_t = ss_t + r[3]
        ce_sel = ce_sel + r[4]
        hint = hint + r[5]

    lane128 = jax.lax.broadcasted_iota(jnp.int32, (1, 128), 1)

    def put(k, v):
        return jnp.where(lane128 == k, v, 0.0)

    vals = (put(0, n_pos) + put(1, l_l) + put(2, ss_s) + put(3, ss_t)
            + put(4, ce_sel) + put(5, hint))
    acc = jnp.where(b == 0, vals, out_ref[...] + vals)

    def get(k):
        return jnp.sum(jnp.where(lane128 == k, acc, 0.0))

    Nf = get(0)
    llT = get(1)
    mse_s = get(2) / (Nf * 4.0)
    mse_t = get(3) / (Nf * 4.0)
    ceT = get(4)
    hintT = get(5)
    lbr = jnp.where(mse_s > mse_t, 0.5 * mse_s, 0.0)
    o1 = (ceT + llT + lbr) / Nf + 0.5 * hintT / hint_denom
    o2 = (ceT + llT) / Nf
    acc = jnp.where(b == nb - 1, acc + put(6, o1) + put(7, o2), acc)
    out_ref[...] = acc


def kernel(loc_data, conf_data, locT, confT, priors, targets):
    B, P, C = conf_data.shape
    O = targets.shape[1]
    locR = jnp.transpose(loc_data, (0, 2, 1))
    locTR = jnp.transpose(locT, (0, 2, 1))
    priT = jnp.transpose(priors, (1, 0))
    imgs = 2 if B % 2 == 0 else 1
    body = functools.partial(_mbody, hint_denom=float(B * P * C), imgs=imgs)
    res = pl.pallas_call(
        body,
        grid=(B // imgs,),
        in_specs=[
            pl.BlockSpec((imgs, 4, P), lambda b: (b, 0, 0)),
            pl.BlockSpec((imgs, P, C), lambda b: (b, 0, 0)),
            pl.BlockSpec((imgs, 4, P), lambda b: (b, 0, 0)),
            pl.BlockSpec((imgs, P, C), lambda b: (b, 0, 0)),
            pl.BlockSpec((4, P), lambda b: (0, 0)),
            pl.BlockSpec((imgs, O, 5), lambda b: (b, 0, 0)),
        ],
        out_specs=pl.BlockSpec((1, 128), lambda b: (0, 0)),
        out_shape=jax.ShapeDtypeStruct((1, 128), jnp.float32),
        compiler_params=pltpu.CompilerParams(
            dimension_semantics=("arbitrary",)),
    )(locR, conf_data, locTR, confT, priT, targets)
    return (res[0, 6], res[0, 7])


# speedup vs baseline: 4.1461x; 1.5058x over previous
import functools

import jax
import jax.numpy as jnp
from jax.experimental import pallas as pl
from jax.experimental.pallas import tpu as pltpu

_THRESH = 0.5
_MAX_FINITE_BITS = 0x7F7FFFFF
_S = 8
_L = 1092


def _smooth_l1(x):
    ax = jnp.abs(x)
    return jnp.where(ax < 1.0, 0.5 * x * x, ax - 0.5)


def _row8(col, fill, P):
    r = col.reshape(1, P)
    pad = jnp.full((1, _S * _L - P), fill, col.dtype)
    flat = jnp.concatenate([r, pad], axis=1)
    return jnp.concatenate(
        [flat[:, i * _L:(i + 1) * _L] for i in range(_S)], axis=0)


def _col(row8, P):
    flat = jnp.concatenate(
        [row8[i:i + 1, :] for i in range(_S)], axis=1)
    return flat[:, :P].reshape(P, 1)


def _one_image(cf, cT, ld, lt, pri, tg):
    P = cf.shape[0]
    C = cf.shape[1]
    O = tg.shape[0]

    pid = (jax.lax.broadcasted_iota(jnp.int32, (_S, _L), 0) * _L
           + jax.lax.broadcasted_iota(jnp.int32, (_S, _L), 1))

    pcx = pri[0]
    pcy = pri[1]
    pw = pri[2]
    ph = pri[3]
    bx1 = pcx - pw * 0.5
    by1 = pcy - ph * 0.5
    bx2 = pcx + pw * 0.5
    by2 = pcy + ph * 0.5
    area_b = (bx2 - bx1) * (by2 - by1)

    tx1 = tg[:, 0:1]
    ty1 = tg[:, 1:2]
    tx2 = tg[:, 2:3]
    ty2 = tg[:, 3:4]
    lab = tg[:, 4:5]

    bto = jnp.zeros((_S, _L), jnp.float32) - 1.0
    bti = jnp.zeros((_S, _L), jnp.int32)
    bpis = []
    for j in range(O):
        a = tx1[j:j + 1, 0:1]
        b_ = ty1[j:j + 1, 0:1]
        c_ = tx2[j:j + 1, 0:1]
        d = ty2[j:j + 1, 0:1]
        iw = jnp.maximum(jnp.minimum(c_, bx2) - jnp.maximum(a, bx1), 0.0)
        ih = jnp.maximum(jnp.minimum(d, by2) - jnp.maximum(b_, by1), 0.0)
        inter = iw * ih
        area_a = (c_ - a) * (d - b_)
        ovj = inter / (area_a + area_b - inter)
        better = ovj > bto
        bto = jnp.where(better, ovj, bto)
        bti = jnp.where(better, j, bti)
        bpoj = jnp.max(ovj)
        bpis.append(jnp.min(jnp.where(ovj == bpoj, pid, _S * _L)))

    for j in range(O):
        mj = pid == bpis[j]
        bto = jnp.where(mj, 2.0, bto)
        bti = jnp.where(mj, j, bti)

    labg = jnp.zeros((_S, _L), jnp.float32)
    mx1 = jnp.zeros((_S, _L), jnp.float32)
    my1 = jnp.zeros((_S, _L), jnp.float32)
    mx2 = jnp.zeros((_S, _L), jnp.float32)
    my2 = jnp.zeros((_S, _L), jnp.float32)
    for j in range(O):
        e = bti == j
        labg = jnp.where(e, lab[j:j + 1, 0:1], labg)
        mx1 = jnp.where(e, tx1[j:j + 1, 0:1], mx1)
        my1 = jnp.where(e, ty1[j:j + 1, 0:1], my1)
        mx2 = jnp.where(e, tx2[j:j + 1, 0:1], mx2)
        my2 = jnp.where(e, ty2[j:j + 1, 0:1], my2)

    conf_row = jnp.where(bto < _THRESH, 0, labg.astype(jnp.int32) + 1)
    posr = conf_row > 0
    posf = posr.astype(jnp.float32)
    n_pos = jnp.sum(posf)

    g_cx = ((mx1 + mx2) * 0.5 - pcx) / (0.1 * pw)
    g_cy = ((my1 + my2) * 0.5 - pcy) / (0.1 * ph)
    g_w = jnp.log((mx2 - mx1) / pw) / 0.2
    g_h = jnp.log((my2 - my1) / ph) / 0.2

    l_l = jnp.float32(0.0)
    ss_s = jnp.float32(0.0)
    ss_t = jnp.float32(0.0)
    for k, g in enumerate((g_cx, g_cy, g_w, g_h)):
        ds = ld[k] - g
        dt = lt[k] - g
        l_l = l_l + jnp.sum(_smooth_l1(ds) * posf)
        ss_s = ss_s + jnp.sum(ds * ds * posf)
        ss_t = ss_t + jnp.sum(dt * dt * posf)

    hint = jnp.sum((cf - cT) ** 2)
    se = jnp.sum(jnp.exp(cf), axis=1, keepdims=True)
    ct_col = _col(conf_row, P)
    ccols = jax.lax.broadcasted_iota(jnp.int32, (P, C), 1)
    gathered = jnp.sum(jnp.where(ccols == ct_col, cf, 0.0), axis=1,
                       keepdims=True)
    se8 = _row8(se, 1.0, P)
    g8 = _row8(gathered, 0.0, P)
    ce_row = jnp.log(se8) - g8
    lossc = jnp.where(posr, 0.0, ce_row)

    bits = jax.lax.bitcast_convert_type(lossc, jnp.int32)
    sbits = bits ^ jnp.int32(-2 ** 31)
    K = jnp.minimum(3 * jnp.sum(posr.astype(jnp.int32)), P - 1)
    lo = jnp.int32(-2 ** 31)
    w = _MAX_FINITE_BITS + 1
    for _ in range(11):
        step = -(-w // 8)
        reds = []
        for j in range(1, 8, 2):
            ga = jnp.where(sbits >= lo + jnp.int32(j * step), 1, 0)
            if j + 1 < 8:
                ga = ga + jnp.where(
                    sbits >= lo + jnp.int32((j + 1) * step), 1 << 16, 0)
            reds.append(jnp.sum(ga))
        jmax = jnp.int32(0)
        for i, r in enumerate(reds):
            jmax = jmax + ((r & 0xFFFF) >= K).astype(jnp.int32)
            if 2 * i + 2 < 8:
                jmax = jmax + ((r >> 16) >= K).astype(jnp.int32)
        lo = lo + jmax * jnp.int32(step)
        w = step
    vb = lo
    cnt_gt = jnp.sum(jnp.where(sbits > vb, 1, 0))
    need = K - cnt_gt
    tie = sbits == vb
    lo2 = jnp.int32(0)
    w2 = 16384
    for _ in range(7):
        st = w2 // 4
        g1 = jnp.where(tie & (pid < lo2 + jnp.int32(st)), 1, 0)
        g1 = g1 + jnp.where(tie & (pid < lo2 + jnp.int32(2 * st)),
                            1 << 16, 0)
        r12 = jnp.sum(g1)
        c3 = jnp.sum(jnp.where(tie & (pid < lo2 + jnp.int32(3 * st)), 1, 0))
        jbel = ((r12 & 0xFFFF) < need).astype(jnp.int32)
        jbel = jbel + ((r12 >> 16) < need).astype(jnp.int32)
        jbel = jbel + (c3 < need).astype(jnp.int32)
        lo2 = lo2 + jbel * jnp.int32(st)
        w2 = st
    cfin = jnp.sum(jnp.where(tie & (pid < lo2), 1, 0))
    lo2 = jnp.where(cfin >= need, lo2, lo2 + 1)
    neg = (sbits > vb) | (tie & (pid < lo2))
    sel = (posr | neg).astype(jnp.float32)
    ce_sel = jnp.sum(ce_row * sel)
    return n_pos, l_l, ss_s, ss_t, ce_sel, hint


def _mbody(locR_ref, conf_ref, locTR_ref, confT_ref, priT_ref, tgt_ref,
           out_ref, *, hint_denom, imgs):
    b = pl.program_id(0)
    nb = pl.num_programs(0)
    pri = priT_ref[...]

    n_pos = l_l = ss_s = ss_t = ce_sel = hint = jnp.float32(0.0)
    for i in range(imgs):
        r = _one_image(conf_ref[i], confT_ref[i], locR_ref[i], locTR_ref[i],
                       pri, tgt_ref[i])
        n_pos = n_pos + r[0]
        l_l = l_l + r[1]
        ss_s = ss_s + r[2]
        ss_t = ss_t + r[3]
        ce_sel = ce_sel + r[4]
        hint = hint + r[5]

    lane128 = jax.lax.broadcasted_iota(jnp.int32, (1, 128), 1)

    def put(k, v):
        return jnp.where(lane128 == k, v, 0.0)

    vals = (put(0, n_pos) + put(1, l_l) + put(2, ss_s) + put(3, ss_t)
            + put(4, ce_sel) + put(5, hint))
    acc = jnp.where(b == 0, vals, out_ref[...] + vals)

    def get(k):
        return jnp.sum(jnp.where(lane128 == k, acc, 0.0))

    Nf = get(0)
    llT = get(1)
    mse_s = get(2) / (Nf * 4.0)
    mse_t = get(3) / (Nf * 4.0)
    ceT = get(4)
    hintT = get(5)
    lbr = jnp.where(mse_s > mse_t, 0.5 * mse_s, 0.0)
    o1 = (ceT + llT + lbr) / Nf + 0.5 * hintT / hint_denom
    o2 = (ceT + llT) / Nf
    acc = jnp.where(b == nb - 1, acc + put(6, o1) + put(7, o2), acc)
    out_ref[...] = acc


def kernel(loc_data, conf_data, locT, confT, priors, targets):
    B, P, C = conf_data.shape
    O = targets.shape[1]
    npad = _S * _L - P

    def to_row8(x):
        xt = jnp.transpose(x, (0, 2, 1))
        xt = jnp.pad(xt, ((0, 0), (0, 0), (0, npad)))
        return xt.reshape(B, 4, _S, _L)

    locR = to_row8(loc_data)
    locTR = to_row8(locT)
    priT = jnp.transpose(priors, (1, 0))
    pad_pri = jnp.tile(jnp.array([[-100.0], [-100.0], [1.0], [1.0]],
                                 dtype=priors.dtype), (1, npad))
    priT = jnp.concatenate([priT, pad_pri], axis=1).reshape(4, _S, _L)

    imgs = 2 if B % 2 == 0 else 1
    body = functools.partial(_mbody, hint_denom=float(B * P * C), imgs=imgs)
    res = pl.pallas_call(
        body,
        grid=(B // imgs,),
        in_specs=[
            pl.BlockSpec((imgs, 4, _S, _L), lambda b: (b, 0, 0, 0)),
            pl.BlockSpec((imgs, P, C), lambda b: (b, 0, 0)),
            pl.BlockSpec((imgs, 4, _S, _L), lambda b: (b, 0, 0, 0)),
            pl.BlockSpec((imgs, P, C), lambda b: (b, 0, 0)),
            pl.BlockSpec((4, _S, _L), lambda b: (0, 0, 0)),
            pl.BlockSpec((imgs, O, 5), lambda b: (b, 0, 0)),
        ],
        out_specs=pl.BlockSpec((1, 128), lambda b: (0, 0)),
        out_shape=jax.ShapeDtypeStruct((1, 128), jnp.float32),
        compiler_params=pltpu.CompilerParams(
            dimension_semantics=("arbitrary",)),
    )(locR, conf_data, locTR, confT, priT, targets)
    return (res[0, 6], res[0, 7])
